# baseline (speedup 1.0000x reference)
import functools

import jax
import jax.numpy as jnp
from jax import lax
from jax.experimental import pallas as pl
from jax.experimental.pallas import tpu as pltpu
from jax.experimental.pallas import tpu_sc as plsc

N = 10000
E = 320000
P = 320000
H = 128
DEPTH = 4

NPAD = 10240
NC, NS = 2, 16
NW = NC * NS
KE = 40
PPW = P // NW
KP = 2000

CHUNKS = (158720, 161280)

f32 = jnp.float32
bf16 = jnp.bfloat16
i32 = jnp.int32


def _mesh():
    return plsc.VectorSubcoreMesh(core_axis_name="c", subcore_axis_name="s")


def _pack_bf16(lo, hi):
    def rnd(v):
        r = jax.lax.bitcast_convert_type(v, i32)
        odd = jnp.bitwise_and(jax.lax.shift_right_logical(r, 16), 1)
        return jax.lax.shift_right_logical(r + 32767 + odd, 16)
    return jnp.bitwise_or(rnd(lo), jnp.left_shift(rnd(hi), 16))


def _unpack_bf16(x):
    lo = jax.lax.bitcast_convert_type(jnp.left_shift(x, 16), f32)
    hi = jax.lax.bitcast_convert_type(jnp.bitwise_and(x, jnp.int32(-65536)),
                                      f32)
    return lo, hi


def _sc_gather(T, cidx, cnt):
    epw = cnt // NW
    nbody = epw // (2 * KE)
    run = functools.partial(
        pl.kernel, mesh=_mesh(),
        out_type=(jax.ShapeDtypeStruct((cnt, H), f32),
                  jax.ShapeDtypeStruct((cnt, H), f32)),
        scratch_types=[
            pltpu.VMEM((2 * KE,), i32), pltpu.VMEM((2 * KE,), i32),
            pltpu.VMEM((2 * KE, H), f32), pltpu.VMEM((2 * KE, H), f32),
            pltpu.SemaphoreType.DMA, pltpu.SemaphoreType.DMA,
            pltpu.SemaphoreType.DMA, pltpu.SemaphoreType.DMA,
            pltpu.SemaphoreType.DMA, pltpu.SemaphoreType.DMA,
        ])

    @run
    def body(t_hbm, ci_hbm, ga_hbm, gb_hbm,
             ix0, ix1, bg0, bg1,
             semi0, semi1, semg0, semg1, semw0, semw1):
        wid = lax.axis_index("s") * NC + lax.axis_index("c")
        base0 = wid * epw
        pltpu.async_copy(ci_hbm.at[pl.ds(2 * base0, 2 * KE)], ix0, semi0)
        pltpu.async_copy(ci_hbm.at[pl.ds(2 * (base0 + KE), 2 * KE)], ix1,
                         semi1)

        def step(cc, carry):
            c0 = base0 + cc * (2 * KE)
            c1 = c0 + KE

            @pl.when(cc > 0)
            def _():
                pltpu.make_async_copy(bg0, ga_hbm.at[pl.ds(base0, 2 * KE)],
                                      semw0).wait()

            pltpu.make_async_copy(ci_hbm.at[pl.ds(2 * base0, 2 * KE)], ix0,
                                  semi0).wait()
            g0 = pltpu.async_copy(t_hbm.at[ix0], bg0, semg0)

            @pl.when(cc > 0)
            def _():
                pltpu.make_async_copy(bg1, ga_hbm.at[pl.ds(base0, 2 * KE)],
                                      semw1).wait()

            pltpu.make_async_copy(ci_hbm.at[pl.ds(2 * base0, 2 * KE)], ix1,
                                  semi1).wait()
            g1 = pltpu.async_copy(t_hbm.at[ix1], bg1, semg1)

            g0.wait()
            pltpu.async_copy(bg0.at[pl.ds(0, KE)],
                             ga_hbm.at[pl.ds(c0, KE)], semw0)
            pltpu.async_copy(bg0.at[pl.ds(KE, KE)],
                             gb_hbm.at[pl.ds(c0, KE)], semw0)

            @pl.when(cc < nbody - 1)
            def _():
                nxt = c0 + 2 * KE
                pltpu.async_copy(ci_hbm.at[pl.ds(2 * nxt, 2 * KE)], ix0,
                                 semi0)

            g1.wait()
            pltpu.async_copy(bg1.at[pl.ds(0, KE)],
                             ga_hbm.at[pl.ds(c1, KE)], semw1)
            pltpu.async_copy(bg1.at[pl.ds(KE, KE)],
                             gb_hbm.at[pl.ds(c1, KE)], semw1)

            @pl.when(cc < nbody - 1)
            def _():
                nxt1 = c1 + 2 * KE
                pltpu.async_copy(ci_hbm.at[pl.ds(2 * nxt1, 2 * KE)], ix1,
                                 semi1)
            return carry

        lax.fori_loop(0, nbody, step, 0)
        pltpu.make_async_copy(bg0, ga_hbm.at[pl.ds(base0, 2 * KE)],
                              semw0).wait()
        pltpu.make_async_copy(bg1, ga_hbm.at[pl.ds(base0, 2 * KE)],
                              semw1).wait()

    return body(T, cidx)


def _sc_scatter(msg, dst, zeros_nodes, cnt):
    epw = cnt // NW
    nbody = epw // (2 * KE)
    ROWS = NPAD // NS

    run = functools.partial(
        pl.kernel, mesh=_mesh(),
        out_type=jax.ShapeDtypeStruct((2 * NPAD, H), f32),
        scratch_types=[
            pltpu.VMEM((KE,), i32), pltpu.VMEM((KE,), i32),
            pltpu.VMEM((KE, H), f32), pltpu.VMEM((KE, H), f32),
            pltpu.VMEM_SHARED((NPAD, H), f32),
            pltpu.SemaphoreType.DMA, pltpu.SemaphoreType.DMA,
            pltpu.SemaphoreType.DMA, pltpu.SemaphoreType.DMA,
        ])

    @run
    def body(msg_hbm, dst_hbm, z_hbm, out_hbm, id0, id1, bm0, bm1v, acc,
             seml0, seml1, semc0, semc1):
        cid = lax.axis_index("c")
        sid = lax.axis_index("s")
        wid = sid * NC + cid
        pltpu.sync_copy(z_hbm.at[pl.ds(sid * ROWS, ROWS)],
                        acc.at[pl.ds(sid * ROWS, ROWS)])
        plsc.subcore_barrier()
        base0 = wid * epw
        pltpu.async_copy(dst_hbm.at[pl.ds(base0, KE)], id0, seml0)
        pltpu.async_copy(msg_hbm.at[pl.ds(base0, KE)], bm0, seml0)

        pltpu.async_copy(dst_hbm.at[pl.ds(base0 + KE, KE)], id1, seml1)
        pltpu.async_copy(msg_hbm.at[pl.ds(base0 + KE, KE)], bm1v, seml1)

        def step(cc, carry):
            c0 = base0 + cc * (2 * KE)
            c1 = c0 + KE
            pltpu.make_async_copy(dst_hbm.at[pl.ds(base0, KE)], id0,
                                  seml0).wait()
            pltpu.make_async_copy(msg_hbm.at[pl.ds(base0, KE)], bm0,
                                  seml0).wait()
            h0 = pltpu.async_copy(bm0, acc.at[id0], semc0, add=True)
            pltpu.make_async_copy(dst_hbm.at[pl.ds(base0, KE)], id1,
                                  seml1).wait()
            pltpu.make_async_copy(msg_hbm.at[pl.ds(base0, KE)], bm1v,
                                  seml1).wait()
            h1 = pltpu.async_copy(bm1v, acc.at[id1], semc1, add=True)
            h0.wait()

            @pl.when(cc < nbody - 1)
            def _():
                nxt = c0 + 2 * KE
                pltpu.async_copy(dst_hbm.at[pl.ds(nxt, KE)], id0, seml0)
                pltpu.async_copy(msg_hbm.at[pl.ds(nxt, KE)], bm0, seml0)

            h1.wait()

            @pl.when(cc < nbody - 1)
            def _():
                nxt1 = c1 + 2 * KE
                pltpu.async_copy(dst_hbm.at[pl.ds(nxt1, KE)], id1, seml1)
                pltpu.async_copy(msg_hbm.at[pl.ds(nxt1, KE)], bm1v, seml1)
            return carry

        lax.fori_loop(0, nbody, step, 0)
        plsc.subcore_barrier()
        pltpu.sync_copy(acc.at[pl.ds(sid * ROWS, ROWS)],
                        out_hbm.at[pl.ds(cid * NPAD + sid * ROWS, ROWS)])

    return body(msg, dst, zeros_nodes)


def _sc_head_gather(s, p0, p1):
    run = functools.partial(
        pl.kernel, mesh=_mesh(),
        out_type=jax.ShapeDtypeStruct((P,), f32),
        compiler_params=pltpu.CompilerParams(needs_layout_passes=False),
        scratch_types=[
            pltpu.VMEM((NPAD,), f32),
            pltpu.VMEM((KP,), i32), pltpu.VMEM((KP,), i32),
            pltpu.VMEM((KP,), f32),
        ])

    @run
    def body(s_hbm, p0_hbm, p1_hbm, t_hbm, sv, i0, i1, ov):
        wid = lax.axis_index("s") * NC + lax.axis_index("c")
        pltpu.sync_copy(s_hbm, sv)
        base0 = wid * PPW

        def chunk(c, carry):
            base = base0 + c * KP
            pltpu.sync_copy(p0_hbm.at[pl.ds(base, KP)], i0)
            pltpu.sync_copy(p1_hbm.at[pl.ds(base, KP)], i1)

            def inner(j, carry2):
                ix0 = i0[pl.ds(j * 16, 16)]
                ix1 = i1[pl.ds(j * 16, 16)]
                g0 = plsc.load_gather(sv, [ix0])
                g1 = plsc.load_gather(sv, [ix1])
                ov[pl.ds(j * 16, 16)] = g0 + g1
                return carry2

            lax.fori_loop(0, KP // 16, inner, 0)
            pltpu.sync_copy(ov, t_hbm.at[pl.ds(base, KP)])
            return carry

        lax.fori_loop(0, PPW // KP, chunk, 0)

    return body(s, p0, p1)


_NBLK = 20
_NB = 1280


def _edge0_call(edge_d, ew, eb, Wc, Wm2_, wsr, b1c, b2, bsc, cnt):
    blk = cnt // _NBLK
    row = lambda idx: (idx, 0)
    zero = lambda idx: (0, 0)

    def body(ed_r, ew_r, eb_r, wc_r, wm2_r, ws_r, b1_r, b2_r, bs_r,
             dn_r, msg_r):
        d0 = jax.nn.silu(jax.nn.silu(ed_r[...] * ew_r[...] + eb_r[...]))
        m1 = jax.nn.silu(
            jnp.dot(d0, wc_r[...], preferred_element_type=f32) + b1_r[...])
        m2 = jax.nn.silu(
            jnp.dot(m1, wm2_r[...], preferred_element_type=f32) + b2_r[...])
        dn_r[...] = d0 + m2
        wl = jnp.sum(m2 * ws_r[...], axis=1, keepdims=True) + bs_r[...]
        msg_r[...] = m2 * jax.nn.sigmoid(wl)

    return pl.pallas_call(
        body, grid=(_NBLK,),
        in_specs=[pl.BlockSpec((blk, 1), row),
                  pl.BlockSpec((1, H), zero), pl.BlockSpec((1, H), zero),
                  pl.BlockSpec((H, H), zero), pl.BlockSpec((H, H), zero),
                  pl.BlockSpec((1, H), zero), pl.BlockSpec((1, H), zero),
                  pl.BlockSpec((1, H), zero), pl.BlockSpec((1, 1), zero)],
        out_specs=[pl.BlockSpec((blk, H), row), pl.BlockSpec((blk, H), row)],
        out_shape=[jax.ShapeDtypeStruct((cnt, H), f32),
                   jax.ShapeDtypeStruct((cnt, H), f32)],
    )(edge_d, ew, eb, Wc, Wm2_, wsr, b1c, b2, bsc)


def _edge_call(ga, gb, d, Wc, Wm2_, wsr, b1, b2, bsc, cnt):
    blk = cnt // _NBLK
    row = lambda idx: (idx, 0)
    zero = lambda idx: (0, 0)

    def body(ga_r, gb_r, d_r, wc_r, wm2_r, ws_r, b1_r, b2_r, bs_r,
             dn_r, msg_r):
        dv = d_r[...]
        m1 = jax.nn.silu(
            ga_r[...] + gb_r[...]
            + jnp.dot(dv, wc_r[...], preferred_element_type=f32) + b1_r[...])
        m2 = jax.nn.silu(
            jnp.dot(m1, wm2_r[...], preferred_element_type=f32) + b2_r[...])
        dn_r[...] = dv + m2
        wl = jnp.sum(m2 * ws_r[...], axis=1, keepdims=True) + bs_r[...]
        msg_r[...] = m2 * jax.nn.sigmoid(wl)

    return pl.pallas_call(
        body, grid=(_NBLK,),
        in_specs=[pl.BlockSpec((blk, H), row), pl.BlockSpec((blk, H), row),
                  pl.BlockSpec((blk, H), row),
                  pl.BlockSpec((H, H), zero), pl.BlockSpec((H, H), zero),
                  pl.BlockSpec((1, H), zero), pl.BlockSpec((1, H), zero),
                  pl.BlockSpec((1, H), zero), pl.BlockSpec((1, 1), zero)],
        out_specs=[pl.BlockSpec((blk, H), row), pl.BlockSpec((blk, H), row)],
        out_shape=[jax.ShapeDtypeStruct((cnt, H), f32),
                   jax.ShapeDtypeStruct((cnt, H), f32)],
    )(ga, gb, d, Wc, Wm2_, wsr, b1, b2, bsc)


def _update_call(h, parts, Wu1_, bu1_, Wu2_, bu2_, Wa_n, Wb_n):
    grid = (NPAD // _NB,)
    nblocks = NPAD // _NB
    np_ = len(parts)
    row = lambda idx: (idx, 0)
    row2 = lambda idx: (idx + nblocks, 0)
    zero = lambda idx: (0, 0)

    def body(h_r, *rest):
        q_rs = rest[:2 * np_]
        wu1_r, bu1_r, wu2_r, bu2_r, wa_r, wb_r = rest[2 * np_:-2]
        hn_r, t_r = rest[-2:]
        x = h_r[...]
        for q_r in q_rs:
            x = x + q_r[...]
        u = jnp.dot(
            jax.nn.silu(
                jnp.dot(x, wu1_r[...], preferred_element_type=f32)
                + bu1_r[...]),
            wu2_r[...], preferred_element_type=f32) + bu2_r[...]
        hn = h_r[...] + u
        hn_r[...] = hn
        av = jnp.dot(hn, wa_r[...], preferred_element_type=f32)
        bv = jnp.dot(hn, wb_r[...], preferred_element_type=f32)
        t_r[...] = jnp.stack([av, bv], axis=1).reshape(2 * _NB, H)

    return pl.pallas_call(
        body, grid=grid,
        in_specs=[pl.BlockSpec((_NB, H), row)]
                 + [pl.BlockSpec((_NB, H), rr)
                    for _ in parts for rr in (row, row2)]
                 + [pl.BlockSpec((H, H), zero), pl.BlockSpec((1, H), zero),
                    pl.BlockSpec((H, H), zero), pl.BlockSpec((1, H), zero),
                    pl.BlockSpec((H, H), zero), pl.BlockSpec((H, H), zero)],
        out_specs=[pl.BlockSpec((_NB, H), row),
                   pl.BlockSpec((2 * _NB, H), row)],
        out_shape=[jax.ShapeDtypeStruct((NPAD, H), f32),
                   jax.ShapeDtypeStruct((2 * NPAD, H), f32)],
    )(h, *[p for p in parts for _ in (0, 1)],
      Wu1_, bu1_, Wu2_, bu2_, Wa_n, Wb_n)


def _update_last_call(h, parts, Wu1_, bu1_, Wu2_, bu2_, wd_row):
    grid = (NPAD // _NB,)
    nblocks = NPAD // _NB
    np_ = len(parts)
    row = lambda idx: (idx, 0)
    row2 = lambda idx: (idx + nblocks, 0)
    zero = lambda idx: (0, 0)

    def body(h_r, *rest):
        q_rs = rest[:2 * np_]
        wu1_r, bu1_r, wu2_r, bu2_r, wd_r = rest[2 * np_:-1]
        s_r = rest[-1]
        x = h_r[...]
        for q_r in q_rs:
            x = x + q_r[...]
        u = jnp.dot(
            jax.nn.silu(
                jnp.dot(x, wu1_r[...], preferred_element_type=f32)
                + bu1_r[...]),
            wu2_r[...], preferred_element_type=f32) + bu2_r[...]
        hn = h_r[...] + u
        s_r[...] = jnp.sum(hn * wd_r[...], axis=1, keepdims=True)

    return pl.pallas_call(
        body, grid=grid,
        in_specs=[pl.BlockSpec((_NB, H), row)]
                 + [pl.BlockSpec((_NB, H), rr)
                    for _ in parts for rr in (row, row2)]
                 + [pl.BlockSpec((H, H), zero), pl.BlockSpec((1, H), zero),
                    pl.BlockSpec((H, H), zero), pl.BlockSpec((1, H), zero),
                    pl.BlockSpec((1, H), zero)],
        out_specs=pl.BlockSpec((_NB, 1), row),
        out_shape=jax.ShapeDtypeStruct((NPAD, 1), f32),
    )(h, *[p for p in parts for _ in (0, 1)],
      Wu1_, bu1_, Wu2_, bu2_, wd_row)


def _softplus_call(t2, bd2):
    rows = P // H

    def body(t_r, b_r, o_r):
        o_r[...] = jax.nn.softplus(t_r[...] + b_r[...])

    return pl.pallas_call(
        body, grid=(1,),
        in_specs=[pl.BlockSpec((rows, H), lambda idx: (0, 0)),
                  pl.BlockSpec((1, 1), lambda idx: (0, 0))],
        out_specs=pl.BlockSpec((rows, H), lambda idx: (0, 0)),
        out_shape=jax.ShapeDtypeStruct((rows, H), f32),
    )(t2, bd2)


def kernel(edge_index, edge_d, pairwise_indices, mask, node_embedding,
           edge_W, edge_b, Wm1, bm1, Wm2, bm2, Ws, bs, Wu1, bu1, Wu2, bu2,
           Wd, bd):
    src = edge_index[0].astype(i32)
    dst = edge_index[1].astype(i32)
    p0 = pairwise_indices[0].astype(i32)
    p1 = pairwise_indices[1].astype(i32)

    offs = [0]
    for c in CHUNKS[:-1]:
        offs.append(offs[-1] + c)
    halves = tuple(zip(offs, CHUNKS))
    dsts = tuple(lax.slice(dst, (o,), (o + c,)) for o, c in halves)
    eds = tuple(lax.slice(edge_d, (o, 0), (o + c, 1)) for o, c in halves)
    cidx = jnp.concatenate(
        [(2 * src).reshape(E // KE, KE), (2 * dst + 1).reshape(E // KE, KE)],
        axis=1).reshape(2 * E)
    cidxs = tuple(lax.slice(cidx, (2 * o,), (2 * (o + c),)) for o, c in halves)

    zeros_nodes = jnp.zeros((NPAD, H), f32)
    h = jnp.broadcast_to(node_embedding[None, :], (NPAD, H))

    c0 = node_embedding @ Wm1[0, :H] + node_embedding @ Wm1[0, H:2 * H]
    wd_row = (Wd[:H, 0] + Wd[H:, 0]).reshape(1, H)

    T = None
    nh = len(halves)
    d_cur = [None] * nh
    s = None
    for i in range(DEPTH):
        Wc = Wm1[i, 2 * H:]
        b2 = bm2[i].reshape(1, H)
        bsc = bs[i].reshape(1, 1)
        wsr = Ws[i].reshape(1, H)
        msgs = [None] * nh
        if i == 0:
            for k, (_, cnt) in enumerate(halves):
                d_cur[k], msgs[k] = _edge0_call(
                    eds[k], edge_W, edge_b.reshape(1, H), Wc, Wm2[0], wsr,
                    (c0 + bm1[0]).reshape(1, H), b2, bsc, cnt)
        else:
            gs = [_sc_gather(T, cidxs[k], cnt)
                  for k, (_, cnt) in enumerate(halves)]
            for k, (_, cnt) in enumerate(halves):
                d_cur[k], msgs[k] = _edge_call(
                    gs[k][0], gs[k][1], d_cur[k], Wc, Wm2[i], wsr,
                    bm1[i].reshape(1, H), b2, bsc, cnt)
        parts = [_sc_scatter(msgs[k], dsts[k], zeros_nodes, cnt)
                 for k, (_, cnt) in enumerate(halves)]
        if i < DEPTH - 1:
            h, T = _update_call(
                h, parts, Wu1[i], bu1[i].reshape(1, H), Wu2[i],
                bu2[i].reshape(1, H), Wm1[i + 1, :H], Wm1[i + 1, H:2 * H])
        else:
            s = _update_last_call(
                h, parts, Wu1[i], bu1[i].reshape(1, H), Wu2[i],
                bu2[i].reshape(1, H), wd_row)

    t = _sc_head_gather(s.reshape(NPAD), p0, p1)
    dist = _softplus_call(t.reshape(P // H, H), (2.0 * bd).reshape(1, 1))
    return dist.reshape(P, 1)


# speedup vs baseline: 1.0041x; 1.0041x over previous
import functools

import jax
import jax.numpy as jnp
from jax import lax
from jax.experimental import pallas as pl
from jax.experimental.pallas import tpu as pltpu
from jax.experimental.pallas import tpu_sc as plsc

N = 10000
E = 320000
P = 320000
H = 128
DEPTH = 4

NPAD = 10240
NC, NS = 2, 16
NW = NC * NS
KE = 40
PPW = P // NW
KP = 2000

CHUNKS = (158720, 161280)

f32 = jnp.float32
bf16 = jnp.bfloat16
i32 = jnp.int32


def _mesh():
    return plsc.VectorSubcoreMesh(core_axis_name="c", subcore_axis_name="s")


def _pack_bf16(lo, hi):
    def rnd(v):
        r = jax.lax.bitcast_convert_type(v, i32)
        odd = jnp.bitwise_and(jax.lax.shift_right_logical(r, 16), 1)
        return jax.lax.shift_right_logical(r + 32767 + odd, 16)
    return jnp.bitwise_or(rnd(lo), jnp.left_shift(rnd(hi), 16))


def _unpack_bf16(x):
    lo = jax.lax.bitcast_convert_type(jnp.left_shift(x, 16), f32)
    hi = jax.lax.bitcast_convert_type(jnp.bitwise_and(x, jnp.int32(-65536)),
                                      f32)
    return lo, hi


def _sc_gather(A, B, src, dst, cnt):
    epw = cnt // NW
    nbody = epw // (2 * KE)
    run = functools.partial(
        pl.kernel, mesh=_mesh(),
        out_type=(jax.ShapeDtypeStruct((cnt, H), f32),
                  jax.ShapeDtypeStruct((cnt, H), f32)),
        scratch_types=[
            pltpu.VMEM((KE,), i32), pltpu.VMEM((KE,), i32),
            pltpu.VMEM((KE,), i32), pltpu.VMEM((KE,), i32),
            pltpu.VMEM((KE, H), f32), pltpu.VMEM((KE, H), f32),
            pltpu.VMEM((KE, H), f32), pltpu.VMEM((KE, H), f32),
            pltpu.SemaphoreType.DMA, pltpu.SemaphoreType.DMA,
            pltpu.SemaphoreType.DMA, pltpu.SemaphoreType.DMA,
            pltpu.SemaphoreType.DMA, pltpu.SemaphoreType.DMA,
        ])

    @run
    def body(a_hbm, b_hbm, src_hbm, dst_hbm, ga_hbm, gb_hbm,
             is0, id0, is1, id1, ba0, bb0, ba1, bb1,
             semi0, semi1, semg0, semg1, semw0, semw1):
        wid = lax.axis_index("s") * NC + lax.axis_index("c")
        base0 = wid * epw
        pltpu.async_copy(src_hbm.at[pl.ds(base0, KE)], is0, semi0)
        pltpu.async_copy(dst_hbm.at[pl.ds(base0, KE)], id0, semi0)
        pltpu.async_copy(src_hbm.at[pl.ds(base0 + KE, KE)], is1, semi1)
        pltpu.async_copy(dst_hbm.at[pl.ds(base0 + KE, KE)], id1, semi1)

        def step(cc, carry):
            c0 = base0 + cc * (2 * KE)
            c1 = c0 + KE

            @pl.when(cc > 0)
            def _():
                pltpu.make_async_copy(ba0, ga_hbm.at[pl.ds(base0, KE)],
                                      semw0).wait()
                pltpu.make_async_copy(bb0, gb_hbm.at[pl.ds(base0, KE)],
                                      semw0).wait()

            pltpu.make_async_copy(src_hbm.at[pl.ds(base0, KE)], is0,
                                  semi0).wait()
            pltpu.make_async_copy(dst_hbm.at[pl.ds(base0, KE)], id0,
                                  semi0).wait()
            g0a = pltpu.async_copy(a_hbm.at[is0], ba0, semg0)
            g0b = pltpu.async_copy(b_hbm.at[id0], bb0, semg0)

            @pl.when(cc > 0)
            def _():
                pltpu.make_async_copy(ba1, ga_hbm.at[pl.ds(base0, KE)],
                                      semw1).wait()
                pltpu.make_async_copy(bb1, gb_hbm.at[pl.ds(base0, KE)],
                                      semw1).wait()

            pltpu.make_async_copy(src_hbm.at[pl.ds(base0, KE)], is1,
                                  semi1).wait()
            pltpu.make_async_copy(dst_hbm.at[pl.ds(base0, KE)], id1,
                                  semi1).wait()
            g1a = pltpu.async_copy(a_hbm.at[is1], ba1, semg1)
            g1b = pltpu.async_copy(b_hbm.at[id1], bb1, semg1)

            g0a.wait()
            g0b.wait()
            pltpu.async_copy(ba0, ga_hbm.at[pl.ds(c0, KE)], semw0)
            pltpu.async_copy(bb0, gb_hbm.at[pl.ds(c0, KE)], semw0)

            @pl.when(cc < nbody - 1)
            def _():
                nxt = c0 + 2 * KE
                pltpu.async_copy(src_hbm.at[pl.ds(nxt, KE)], is0, semi0)
                pltpu.async_copy(dst_hbm.at[pl.ds(nxt, KE)], id0, semi0)

            g1a.wait()
            g1b.wait()
            pltpu.async_copy(ba1, ga_hbm.at[pl.ds(c1, KE)], semw1)
            pltpu.async_copy(bb1, gb_hbm.at[pl.ds(c1, KE)], semw1)

            @pl.when(cc < nbody - 1)
            def _():
                nxt1 = c1 + 2 * KE
                pltpu.async_copy(src_hbm.at[pl.ds(nxt1, KE)], is1, semi1)
                pltpu.async_copy(dst_hbm.at[pl.ds(nxt1, KE)], id1, semi1)
            return carry

        lax.fori_loop(0, nbody, step, 0)
        pltpu.make_async_copy(ba0, ga_hbm.at[pl.ds(base0, KE)], semw0).wait()
        pltpu.make_async_copy(bb0, gb_hbm.at[pl.ds(base0, KE)], semw0).wait()
        pltpu.make_async_copy(ba1, ga_hbm.at[pl.ds(base0, KE)], semw1).wait()
        pltpu.make_async_copy(bb1, gb_hbm.at[pl.ds(base0, KE)], semw1).wait()

    return body(A, B, src, dst)


def _sc_scatter(msg, dst, zeros_nodes, cnt):
    epw = cnt // NW
    nbody = epw // (2 * KE)
    ROWS = NPAD // NS

    run = functools.partial(
        pl.kernel, mesh=_mesh(),
        out_type=jax.ShapeDtypeStruct((2 * NPAD, H), f32),
        scratch_types=[
            pltpu.VMEM((KE,), i32), pltpu.VMEM((KE,), i32),
            pltpu.VMEM((KE, H), f32), pltpu.VMEM((KE, H), f32),
            pltpu.VMEM_SHARED((NPAD, H), f32),
            pltpu.SemaphoreType.DMA, pltpu.SemaphoreType.DMA,
            pltpu.SemaphoreType.DMA, pltpu.SemaphoreType.DMA,
        ])

    @run
    def body(msg_hbm, dst_hbm, z_hbm, out_hbm, id0, id1, bm0, bm1v, acc,
             seml0, seml1, semc0, semc1):
        cid = lax.axis_index("c")
        sid = lax.axis_index("s")
        wid = sid * NC + cid
        pltpu.sync_copy(z_hbm.at[pl.ds(sid * ROWS, ROWS)],
                        acc.at[pl.ds(sid * ROWS, ROWS)])
        plsc.subcore_barrier()
        base0 = wid * epw
        pltpu.async_copy(dst_hbm.at[pl.ds(base0, KE)], id0, seml0)
        pltpu.async_copy(msg_hbm.at[pl.ds(base0, KE)], bm0, seml0)

        pltpu.async_copy(dst_hbm.at[pl.ds(base0 + KE, KE)], id1, seml1)
        pltpu.async_copy(msg_hbm.at[pl.ds(base0 + KE, KE)], bm1v, seml1)

        def step(cc, carry):
            c0 = base0 + cc * (2 * KE)
            c1 = c0 + KE
            pltpu.make_async_copy(dst_hbm.at[pl.ds(base0, KE)], id0,
                                  seml0).wait()
            pltpu.make_async_copy(msg_hbm.at[pl.ds(base0, KE)], bm0,
                                  seml0).wait()
            h0 = pltpu.async_copy(bm0, acc.at[id0], semc0, add=True)
            pltpu.make_async_copy(dst_hbm.at[pl.ds(base0, KE)], id1,
                                  seml1).wait()
            pltpu.make_async_copy(msg_hbm.at[pl.ds(base0, KE)], bm1v,
                                  seml1).wait()
            h1 = pltpu.async_copy(bm1v, acc.at[id1], semc1, add=True)
            h0.wait()

            @pl.when(cc < nbody - 1)
            def _():
                nxt = c0 + 2 * KE
                pltpu.async_copy(dst_hbm.at[pl.ds(nxt, KE)], id0, seml0)
                pltpu.async_copy(msg_hbm.at[pl.ds(nxt, KE)], bm0, seml0)

            h1.wait()

            @pl.when(cc < nbody - 1)
            def _():
                nxt1 = c1 + 2 * KE
                pltpu.async_copy(dst_hbm.at[pl.ds(nxt1, KE)], id1, seml1)
                pltpu.async_copy(msg_hbm.at[pl.ds(nxt1, KE)], bm1v, seml1)
            return carry

        lax.fori_loop(0, nbody, step, 0)
        plsc.subcore_barrier()
        pltpu.sync_copy(acc.at[pl.ds(sid * ROWS, ROWS)],
                        out_hbm.at[pl.ds(cid * NPAD + sid * ROWS, ROWS)])

    return body(msg, dst, zeros_nodes)


def _sc_head_gather(s, p0, p1):
    run = functools.partial(
        pl.kernel, mesh=_mesh(),
        out_type=jax.ShapeDtypeStruct((P,), f32),
        compiler_params=pltpu.CompilerParams(needs_layout_passes=False),
        scratch_types=[
            pltpu.VMEM((NPAD,), f32),
            pltpu.VMEM((KP,), i32), pltpu.VMEM((KP,), i32),
            pltpu.VMEM((KP,), f32),
        ])

    @run
    def body(s_hbm, p0_hbm, p1_hbm, t_hbm, sv, i0, i1, ov):
        wid = lax.axis_index("s") * NC + lax.axis_index("c")
        pltpu.sync_copy(s_hbm, sv)
        base0 = wid * PPW

        def chunk(c, carry):
            base = base0 + c * KP
            pltpu.sync_copy(p0_hbm.at[pl.ds(base, KP)], i0)
            pltpu.sync_copy(p1_hbm.at[pl.ds(base, KP)], i1)

            def inner(j, carry2):
                ix0 = i0[pl.ds(j * 16, 16)]
                ix1 = i1[pl.ds(j * 16, 16)]
                g0 = plsc.load_gather(sv, [ix0])
                g1 = plsc.load_gather(sv, [ix1])
                ov[pl.ds(j * 16, 16)] = g0 + g1
                return carry2

            lax.fori_loop(0, KP // 16, inner, 0)
            pltpu.sync_copy(ov, t_hbm.at[pl.ds(base, KP)])
            return carry

        lax.fori_loop(0, PPW // KP, chunk, 0)

    return body(s, p0, p1)


_NBLK = 20
_NB = 1280


def _edge0_call(edge_d, ew, eb, Wc, Wm2_, wsr, b1c, b2, bsc, cnt):
    blk = cnt // _NBLK
    row = lambda idx: (idx, 0)
    zero = lambda idx: (0, 0)

    def body(ed_r, ew_r, eb_r, wc_r, wm2_r, ws_r, b1_r, b2_r, bs_r,
             dn_r, msg_r):
        d0 = jax.nn.silu(jax.nn.silu(ed_r[...] * ew_r[...] + eb_r[...]))
        m1 = jax.nn.silu(
            jnp.dot(d0, wc_r[...], preferred_element_type=f32) + b1_r[...])
        m2 = jax.nn.silu(
            jnp.dot(m1, wm2_r[...], preferred_element_type=f32) + b2_r[...])
        dn_r[...] = d0 + m2
        wl = jnp.sum(m2 * ws_r[...], axis=1, keepdims=True) + bs_r[...]
        msg_r[...] = m2 * jax.nn.sigmoid(wl)

    return pl.pallas_call(
        body, grid=(_NBLK,),
        in_specs=[pl.BlockSpec((blk, 1), row),
                  pl.BlockSpec((1, H), zero), pl.BlockSpec((1, H), zero),
                  pl.BlockSpec((H, H), zero), pl.BlockSpec((H, H), zero),
                  pl.BlockSpec((1, H), zero), pl.BlockSpec((1, H), zero),
                  pl.BlockSpec((1, H), zero), pl.BlockSpec((1, 1), zero)],
        out_specs=[pl.BlockSpec((blk, H), row), pl.BlockSpec((blk, H), row)],
        out_shape=[jax.ShapeDtypeStruct((cnt, H), f32),
                   jax.ShapeDtypeStruct((cnt, H), f32)],
    )(edge_d, ew, eb, Wc, Wm2_, wsr, b1c, b2, bsc)


def _edge_call(ga, gb, d, Wc, Wm2_, wsr, b1, b2, bsc, cnt):
    blk = cnt // _NBLK
    row = lambda idx: (idx, 0)
    zero = lambda idx: (0, 0)

    def body(ga_r, gb_r, d_r, wc_r, wm2_r, ws_r, b1_r, b2_r, bs_r,
             dn_r, msg_r):
        dv = d_r[...]
        m1 = jax.nn.silu(
            ga_r[...] + gb_r[...]
            + jnp.dot(dv, wc_r[...], preferred_element_type=f32) + b1_r[...])
        m2 = jax.nn.silu(
            jnp.dot(m1, wm2_r[...], preferred_element_type=f32) + b2_r[...])
        dn_r[...] = dv + m2
        wl = jnp.sum(m2 * ws_r[...], axis=1, keepdims=True) + bs_r[...]
        msg_r[...] = m2 * jax.nn.sigmoid(wl)

    return pl.pallas_call(
        body, grid=(_NBLK,),
        in_specs=[pl.BlockSpec((blk, H), row), pl.BlockSpec((blk, H), row),
                  pl.BlockSpec((blk, H), row),
                  pl.BlockSpec((H, H), zero), pl.BlockSpec((H, H), zero),
                  pl.BlockSpec((1, H), zero), pl.BlockSpec((1, H), zero),
                  pl.BlockSpec((1, H), zero), pl.BlockSpec((1, 1), zero)],
        out_specs=[pl.BlockSpec((blk, H), row), pl.BlockSpec((blk, H), row)],
        out_shape=[jax.ShapeDtypeStruct((cnt, H), f32),
                   jax.ShapeDtypeStruct((cnt, H), f32)],
    )(ga, gb, d, Wc, Wm2_, wsr, b1, b2, bsc)


def _update_call(h, parts, Wu1_, bu1_, Wu2_, bu2_, Wa_n, Wb_n):
    grid = (NPAD // _NB,)
    nblocks = NPAD // _NB
    np_ = len(parts)
    row = lambda idx: (idx, 0)
    row2 = lambda idx: (idx + nblocks, 0)
    zero = lambda idx: (0, 0)

    def body(h_r, *rest):
        q_rs = rest[:2 * np_]
        wu1_r, bu1_r, wu2_r, bu2_r, wa_r, wb_r = rest[2 * np_:-3]
        hn_r, a_r, b_r = rest[-3:]
        x = h_r[...]
        for q_r in q_rs:
            x = x + q_r[...]
        u = jnp.dot(
            jax.nn.silu(
                jnp.dot(x, wu1_r[...], preferred_element_type=f32)
                + bu1_r[...]),
            wu2_r[...], preferred_element_type=f32) + bu2_r[...]
        hn = h_r[...] + u
        hn_r[...] = hn
        a_r[...] = jnp.dot(hn, wa_r[...], preferred_element_type=f32)
        b_r[...] = jnp.dot(hn, wb_r[...], preferred_element_type=f32)

    return pl.pallas_call(
        body, grid=grid,
        in_specs=[pl.BlockSpec((_NB, H), row)]
                 + [pl.BlockSpec((_NB, H), rr)
                    for _ in parts for rr in (row, row2)]
                 + [pl.BlockSpec((H, H), zero), pl.BlockSpec((1, H), zero),
                    pl.BlockSpec((H, H), zero), pl.BlockSpec((1, H), zero),
                    pl.BlockSpec((H, H), zero), pl.BlockSpec((H, H), zero)],
        out_specs=[pl.BlockSpec((_NB, H), row)] * 3,
        out_shape=[jax.ShapeDtypeStruct((NPAD, H), f32)] * 3,
    )(h, *[p for p in parts for _ in (0, 1)],
      Wu1_, bu1_, Wu2_, bu2_, Wa_n, Wb_n)


def _update_last_call(h, parts, Wu1_, bu1_, Wu2_, bu2_, wd_row):
    grid = (NPAD // _NB,)
    nblocks = NPAD // _NB
    np_ = len(parts)
    row = lambda idx: (idx, 0)
    row2 = lambda idx: (idx + nblocks, 0)
    zero = lambda idx: (0, 0)

    def body(h_r, *rest):
        q_rs = rest[:2 * np_]
        wu1_r, bu1_r, wu2_r, bu2_r, wd_r = rest[2 * np_:-1]
        s_r = rest[-1]
        x = h_r[...]
        for q_r in q_rs:
            x = x + q_r[...]
        u = jnp.dot(
            jax.nn.silu(
                jnp.dot(x, wu1_r[...], preferred_element_type=f32)
                + bu1_r[...]),
            wu2_r[...], preferred_element_type=f32) + bu2_r[...]
        hn = h_r[...] + u
        s_r[...] = jnp.sum(hn * wd_r[...], axis=1, keepdims=True)

    return pl.pallas_call(
        body, grid=grid,
        in_specs=[pl.BlockSpec((_NB, H), row)]
                 + [pl.BlockSpec((_NB, H), rr)
                    for _ in parts for rr in (row, row2)]
                 + [pl.BlockSpec((H, H), zero), pl.BlockSpec((1, H), zero),
                    pl.BlockSpec((H, H), zero), pl.BlockSpec((1, H), zero),
                    pl.BlockSpec((1, H), zero)],
        out_specs=pl.BlockSpec((_NB, 1), row),
        out_shape=jax.ShapeDtypeStruct((NPAD, 1), f32),
    )(h, *[p for p in parts for _ in (0, 1)],
      Wu1_, bu1_, Wu2_, bu2_, wd_row)


def _softplus_call(t2, bd2):
    rows = P // H

    def body(t_r, b_r, o_r):
        o_r[...] = jax.nn.softplus(t_r[...] + b_r[...])

    return pl.pallas_call(
        body, grid=(1,),
        in_specs=[pl.BlockSpec((rows, H), lambda idx: (0, 0)),
                  pl.BlockSpec((1, 1), lambda idx: (0, 0))],
        out_specs=pl.BlockSpec((rows, H), lambda idx: (0, 0)),
        out_shape=jax.ShapeDtypeStruct((rows, H), f32),
    )(t2, bd2)


def kernel(edge_index, edge_d, pairwise_indices, mask, node_embedding,
           edge_W, edge_b, Wm1, bm1, Wm2, bm2, Ws, bs, Wu1, bu1, Wu2, bu2,
           Wd, bd):
    src = edge_index[0].astype(i32)
    dst = edge_index[1].astype(i32)
    p0 = pairwise_indices[0].astype(i32)
    p1 = pairwise_indices[1].astype(i32)

    offs = [0]
    for c in CHUNKS[:-1]:
        offs.append(offs[-1] + c)
    halves = tuple(zip(offs, CHUNKS))
    srcs = tuple(lax.slice(src, (o,), (o + c,)) for o, c in halves)
    dsts = tuple(lax.slice(dst, (o,), (o + c,)) for o, c in halves)
    eds = tuple(lax.slice(edge_d, (o, 0), (o + c, 1)) for o, c in halves)

    zeros_nodes = jnp.zeros((NPAD, H), f32)
    h = jnp.broadcast_to(node_embedding[None, :], (NPAD, H))

    c0 = node_embedding @ Wm1[0, :H] + node_embedding @ Wm1[0, H:2 * H]
    wd_row = (Wd[:H, 0] + Wd[H:, 0]).reshape(1, H)

    A = B = None
    nh = len(halves)
    d_cur = [None] * nh
    s = None
    for i in range(DEPTH):
        Wc = Wm1[i, 2 * H:]
        b2 = bm2[i].reshape(1, H)
        bsc = bs[i].reshape(1, 1)
        wsr = Ws[i].reshape(1, H)
        msgs = [None] * nh
        if i == 0:
            for k, (_, cnt) in enumerate(halves):
                d_cur[k], msgs[k] = _edge0_call(
                    eds[k], edge_W, edge_b.reshape(1, H), Wc, Wm2[0], wsr,
                    (c0 + bm1[0]).reshape(1, H), b2, bsc, cnt)
        else:
            gs = [_sc_gather(A, B, srcs[k], dsts[k], cnt)
                  for k, (_, cnt) in enumerate(halves)]
            for k, (_, cnt) in enumerate(halves):
                d_cur[k], msgs[k] = _edge_call(
                    gs[k][0], gs[k][1], d_cur[k], Wc, Wm2[i], wsr,
                    bm1[i].reshape(1, H), b2, bsc, cnt)
        parts = [_sc_scatter(msgs[k], dsts[k], zeros_nodes, cnt)
                 for k, (_, cnt) in enumerate(halves)]
        if i < DEPTH - 1:
            h, A, B = _update_call(
                h, parts, Wu1[i], bu1[i].reshape(1, H), Wu2[i],
                bu2[i].reshape(1, H), Wm1[i + 1, :H], Wm1[i + 1, H:2 * H])
        else:
            s = _update_last_call(
                h, parts, Wu1[i], bu1[i].reshape(1, H), Wu2[i],
                bu2[i].reshape(1, H), wd_row)

    t = _sc_head_gather(s.reshape(NPAD), p0, p1)
    dist = _softplus_call(t.reshape(P // H, H), (2.0 * bd).reshape(1, 1))
    return dist.reshape(P, 1)
